# baseline (speedup 1.0000x reference)
import functools

import jax
import jax.numpy as jnp
from jax.experimental import pallas as pl
from jax.experimental.pallas import tpu as pltpu

_EPS = 1e-5
_NCH = 8


def _fused_head_kernel(H, W, margin, Pc, MC,
                       xp_ref, mask_ref, wm_ref, bm_ref,
                       w1_ref, b1_ref, w2c_ref, b2c_ref, w2r_ref, b2r_ref,
                       wh_ref, bh_ref,
                       out_ref, fe, cb, rb, hb):
    Wp = W + 2
    P = (H + 2) * Wp
    R = fe.shape[0]
    C = fe.shape[1]
    offs = [margin + (dh - 1) * Wp + (dw - 1)
            for dh in range(3) for dw in range(3)]

    def silu(y):
        return y * jax.lax.logistic(y)

    def zero_margins(ref):
        ref[pl.ds(0, margin), :] = jnp.zeros((margin, C), ref.dtype)
        top = R - margin - Pc
        ref[pl.ds(margin + Pc, top), :] = jnp.zeros((top, C), ref.dtype)

    fe[pl.ds(0, R), :] = jnp.zeros((R, C), fe.dtype)
    zero_margins(cb)
    zero_margins(rb)

    xt = jnp.transpose(xp_ref[0].astype(jnp.bfloat16), (1, 2, 0))
    y = jnp.dot(xt.reshape(H * W, xt.shape[-1]), wm_ref[...],
                preferred_element_type=jnp.float32)
    sb = silu(y + bm_ref[...]).astype(fe.dtype)
    for r in range(H):
        fe[pl.ds(margin + (r + 1) * Wp + 1, W), :] = sb[r * W:(r + 1) * W, :]

    def conv_chunk(src, w_ref, b, c):
        z = jnp.concatenate(
            [src[pl.ds(off + c * MC, MC), :] for off in offs], axis=1)
        acc = jnp.dot(z, w_ref[...], preferred_element_type=jnp.float32)
        return acc + b

    b1 = b1_ref[...]
    for c in range(_NCH):
        m = mask_ref[pl.ds(c * MC, MC), :]
        s = silu(conv_chunk(fe, w1_ref, b1, c)) * m
        cb[pl.ds(margin + c * MC, MC), :] = s[:, :C].astype(cb.dtype)
        rb[pl.ds(margin + c * MC, MC), :] = s[:, C:].astype(rb.dtype)

    b2c, b2r = b2c_ref[...], b2r_ref[...]
    for c in range(_NCH):
        rf = silu(conv_chunk(rb, w2r_ref, b2r, c))
        hb[pl.ds(c * MC, MC), :C] = rf.astype(hb.dtype)
        cf = silu(conv_chunk(cb, w2c_ref, b2c, c))
        hb[pl.ds(c * MC, MC), C:] = cf.astype(hb.dtype)

    xi = jnp.concatenate(
        [hb[pl.ds((r + 1) * Wp + 1, W), :] for r in range(H)], axis=0)
    o = jnp.dot(xi, wh_ref[...], preferred_element_type=jnp.float32)
    o = o + bh_ref[...]
    out_ref[...] = jnp.transpose(o, (1, 0))[None]


def _fold_bn(w_oihw, gamma, beta, mean, var):
    scale = gamma / jnp.sqrt(var + _EPS)
    return w_oihw * scale[:, None, None, None], beta - mean * scale


def _as_1x1(w_oihw):
    return jnp.transpose(w_oihw[:, :, 0, 0], (1, 0))


def _as_taps(w_oihw):
    o, i, _, _ = w_oihw.shape
    return jnp.transpose(w_oihw, (2, 3, 1, 0)).reshape(9 * i, o)


def kernel(x, merge_w, merge_bn_gamma, merge_bn_beta, merge_bn_mean, merge_bn_var,
           cls1_w, cls1_bn_gamma, cls1_bn_beta, cls1_bn_mean, cls1_bn_var,
           cls2_w, cls2_bn_gamma, cls2_bn_beta, cls2_bn_mean, cls2_bn_var,
           reg1_w, reg1_bn_gamma, reg1_bn_beta, reg1_bn_mean, reg1_bn_var,
           reg2_w, reg2_bn_gamma, reg2_bn_beta, reg2_bn_mean, reg2_bn_var,
           cls_pred_w, cls_pred_b, reg_pred_w, reg_pred_b, obj_pred_w, obj_pred_b):
    n, ch, h, w = x.shape
    C = merge_w.shape[0]
    Hp, Wp = h + 2, w + 2
    P = Hp * Wp
    Pc = ((P + 16 * _NCH - 1) // (16 * _NCH)) * (16 * _NCH)
    MC = Pc // _NCH
    margin = ((Wp + 1 + 15) // 16) * 16
    R = margin + Pc + margin
    bf16 = jnp.bfloat16

    ar = jnp.arange(Pc, dtype=jnp.int32)
    hh, ww = ar // Wp, ar % Wp
    interior = ((hh >= 1) & (hh <= h) & (ww >= 1) & (ww <= w) & (ar < P))
    mask = interior.astype(jnp.float32)[:, None]

    wm_f, bm = _fold_bn(merge_w, merge_bn_gamma, merge_bn_beta,
                        merge_bn_mean, merge_bn_var)
    wm = _as_1x1(wm_f).astype(bf16)
    w1c_f, b1c = _fold_bn(cls1_w, cls1_bn_gamma, cls1_bn_beta,
                          cls1_bn_mean, cls1_bn_var)
    w1r_f, b1r = _fold_bn(reg1_w, reg1_bn_gamma, reg1_bn_beta,
                          reg1_bn_mean, reg1_bn_var)
    w1 = jnp.concatenate([_as_taps(w1c_f), _as_taps(w1r_f)], axis=1).astype(bf16)
    b1 = jnp.concatenate([b1c, b1r])[None, :]
    w2c_f, b2c = _fold_bn(cls2_w, cls2_bn_gamma, cls2_bn_beta,
                          cls2_bn_mean, cls2_bn_var)
    w2r_f, b2r = _fold_bn(reg2_w, reg2_bn_gamma, reg2_bn_beta,
                          reg2_bn_mean, reg2_bn_var)
    w2c = _as_taps(w2c_f).astype(bf16)
    w2r = _as_taps(w2r_f).astype(bf16)

    wro = jnp.concatenate([_as_1x1(reg_pred_w), _as_1x1(obj_pred_w)], axis=1)
    wcl = _as_1x1(cls_pred_w)
    nro, ncl = wro.shape[1], wcl.shape[1]
    co = nro + ncl
    wh = jnp.concatenate([
        jnp.concatenate([wro, jnp.zeros((C, ncl), wro.dtype)], axis=1),
        jnp.concatenate([jnp.zeros((C, nro), wcl.dtype), wcl], axis=1),
    ], axis=0).astype(bf16)
    bh = jnp.concatenate([reg_pred_b, obj_pred_b, cls_pred_b])[None, :]

    out = pl.pallas_call(
        functools.partial(_fused_head_kernel, h, w, margin, Pc, MC),
        out_shape=jax.ShapeDtypeStruct((n, co, h * w), jnp.float32),
        grid=(n,),
        in_specs=[
            pl.BlockSpec((1, ch, h, w), lambda i: (i, 0, 0, 0)),
            pl.BlockSpec((Pc, 1), lambda i: (0, 0)),
            pl.BlockSpec((ch, C), lambda i: (0, 0)),
            pl.BlockSpec((1, C), lambda i: (0, 0)),
            pl.BlockSpec((9 * C, 2 * C), lambda i: (0, 0)),
            pl.BlockSpec((1, 2 * C), lambda i: (0, 0)),
            pl.BlockSpec((9 * C, C), lambda i: (0, 0)),
            pl.BlockSpec((1, C), lambda i: (0, 0)),
            pl.BlockSpec((9 * C, C), lambda i: (0, 0)),
            pl.BlockSpec((1, C), lambda i: (0, 0)),
            pl.BlockSpec((2 * C, co), lambda i: (0, 0)),
            pl.BlockSpec((1, co), lambda i: (0, 0)),
        ],
        out_specs=pl.BlockSpec((1, co, h * w), lambda i: (i, 0, 0)),
        scratch_shapes=[pltpu.VMEM((R, C), bf16)] * 3
        + [pltpu.VMEM((Pc, 2 * C), bf16)],
        compiler_params=pltpu.CompilerParams(dimension_semantics=("parallel",)),
    )(x, mask, wm, bm[None, :], w1, b1, w2c, b2c[None, :], w2r, b2r[None, :],
      wh, bh)

    return out.reshape(n, co, h, w)


# speedup vs baseline: 1.0053x; 1.0053x over previous
import functools

import jax
import jax.numpy as jnp
from jax.experimental import pallas as pl
from jax.experimental.pallas import tpu as pltpu

_EPS = 1e-5
_NCH = 6


def _fused_head_kernel(H, W, margin, Pc, MC,
                       xp_ref, mask_ref, wm_ref, bm_ref,
                       w1_ref, b1_ref, w2c_ref, b2c_ref, w2r_ref, b2r_ref,
                       wh_ref, bh_ref,
                       out_ref, fe, cb, rb, hb):
    Wp = W + 2
    P = (H + 2) * Wp
    R = fe.shape[0]
    C = fe.shape[1]
    offs = [margin + (dh - 1) * Wp + (dw - 1)
            for dh in range(3) for dw in range(3)]

    def silu(y):
        return y * jax.lax.logistic(y)

    def zero_margins(ref):
        ref[pl.ds(0, margin), :] = jnp.zeros((margin, C), ref.dtype)
        top = R - margin - Pc
        ref[pl.ds(margin + Pc, top), :] = jnp.zeros((top, C), ref.dtype)

    fe[pl.ds(0, R), :] = jnp.zeros((R, C), fe.dtype)
    zero_margins(cb)
    zero_margins(rb)

    xt = jnp.transpose(xp_ref[0].astype(jnp.bfloat16), (1, 2, 0))
    y = jnp.dot(xt.reshape(H * W, xt.shape[-1]), wm_ref[...],
                preferred_element_type=jnp.float32)
    sb = silu(y + bm_ref[...]).astype(fe.dtype)
    for r in range(H):
        fe[pl.ds(margin + (r + 1) * Wp + 1, W), :] = sb[r * W:(r + 1) * W, :]

    def conv_chunk(src, w_ref, b, c):
        z = jnp.concatenate(
            [src[pl.ds(off + c * MC, MC), :] for off in offs], axis=1)
        acc = jnp.dot(z, w_ref[...], preferred_element_type=jnp.float32)
        return acc + b

    b1 = b1_ref[...]
    for c in range(_NCH):
        m = mask_ref[pl.ds(c * MC, MC), :]
        s = silu(conv_chunk(fe, w1_ref, b1, c)) * m
        cb[pl.ds(margin + c * MC, MC), :] = s[:, :C].astype(cb.dtype)
        rb[pl.ds(margin + c * MC, MC), :] = s[:, C:].astype(rb.dtype)

    b2c, b2r = b2c_ref[...], b2r_ref[...]
    for c in range(_NCH):
        rf = silu(conv_chunk(rb, w2r_ref, b2r, c))
        hb[pl.ds(c * MC, MC), :C] = rf.astype(hb.dtype)
        cf = silu(conv_chunk(cb, w2c_ref, b2c, c))
        hb[pl.ds(c * MC, MC), C:] = cf.astype(hb.dtype)

    xi = jnp.concatenate(
        [hb[pl.ds((r + 1) * Wp + 1, W), :] for r in range(H)], axis=0)
    o = jnp.dot(xi, wh_ref[...], preferred_element_type=jnp.float32)
    o = o + bh_ref[...]
    out_ref[...] = jnp.transpose(o, (1, 0))[None]


def _fold_bn(w_oihw, gamma, beta, mean, var):
    scale = gamma / jnp.sqrt(var + _EPS)
    return w_oihw * scale[:, None, None, None], beta - mean * scale


def _as_1x1(w_oihw):
    return jnp.transpose(w_oihw[:, :, 0, 0], (1, 0))


def _as_taps(w_oihw):
    o, i, _, _ = w_oihw.shape
    return jnp.transpose(w_oihw, (2, 3, 1, 0)).reshape(9 * i, o)


def kernel(x, merge_w, merge_bn_gamma, merge_bn_beta, merge_bn_mean, merge_bn_var,
           cls1_w, cls1_bn_gamma, cls1_bn_beta, cls1_bn_mean, cls1_bn_var,
           cls2_w, cls2_bn_gamma, cls2_bn_beta, cls2_bn_mean, cls2_bn_var,
           reg1_w, reg1_bn_gamma, reg1_bn_beta, reg1_bn_mean, reg1_bn_var,
           reg2_w, reg2_bn_gamma, reg2_bn_beta, reg2_bn_mean, reg2_bn_var,
           cls_pred_w, cls_pred_b, reg_pred_w, reg_pred_b, obj_pred_w, obj_pred_b):
    n, ch, h, w = x.shape
    C = merge_w.shape[0]
    Hp, Wp = h + 2, w + 2
    P = Hp * Wp
    Pc = ((P + 16 * _NCH - 1) // (16 * _NCH)) * (16 * _NCH)
    MC = Pc // _NCH
    margin = ((Wp + 1 + 15) // 16) * 16
    R = margin + Pc + margin
    bf16 = jnp.bfloat16

    ar = jnp.arange(Pc, dtype=jnp.int32)
    hh, ww = ar // Wp, ar % Wp
    interior = ((hh >= 1) & (hh <= h) & (ww >= 1) & (ww <= w) & (ar < P))
    mask = interior.astype(jnp.float32)[:, None]

    wm_f, bm = _fold_bn(merge_w, merge_bn_gamma, merge_bn_beta,
                        merge_bn_mean, merge_bn_var)
    wm = _as_1x1(wm_f).astype(bf16)
    w1c_f, b1c = _fold_bn(cls1_w, cls1_bn_gamma, cls1_bn_beta,
                          cls1_bn_mean, cls1_bn_var)
    w1r_f, b1r = _fold_bn(reg1_w, reg1_bn_gamma, reg1_bn_beta,
                          reg1_bn_mean, reg1_bn_var)
    w1 = jnp.concatenate([_as_taps(w1c_f), _as_taps(w1r_f)], axis=1).astype(bf16)
    b1 = jnp.concatenate([b1c, b1r])[None, :]
    w2c_f, b2c = _fold_bn(cls2_w, cls2_bn_gamma, cls2_bn_beta,
                          cls2_bn_mean, cls2_bn_var)
    w2r_f, b2r = _fold_bn(reg2_w, reg2_bn_gamma, reg2_bn_beta,
                          reg2_bn_mean, reg2_bn_var)
    w2c = _as_taps(w2c_f).astype(bf16)
    w2r = _as_taps(w2r_f).astype(bf16)

    wro = jnp.concatenate([_as_1x1(reg_pred_w), _as_1x1(obj_pred_w)], axis=1)
    wcl = _as_1x1(cls_pred_w)
    nro, ncl = wro.shape[1], wcl.shape[1]
    co = nro + ncl
    wh = jnp.concatenate([
        jnp.concatenate([wro, jnp.zeros((C, ncl), wro.dtype)], axis=1),
        jnp.concatenate([jnp.zeros((C, nro), wcl.dtype), wcl], axis=1),
    ], axis=0).astype(bf16)
    bh = jnp.concatenate([reg_pred_b, obj_pred_b, cls_pred_b])[None, :]

    out = pl.pallas_call(
        functools.partial(_fused_head_kernel, h, w, margin, Pc, MC),
        out_shape=jax.ShapeDtypeStruct((n, co, h * w), jnp.float32),
        grid=(n,),
        in_specs=[
            pl.BlockSpec((1, ch, h, w), lambda i: (i, 0, 0, 0)),
            pl.BlockSpec((Pc, 1), lambda i: (0, 0)),
            pl.BlockSpec((ch, C), lambda i: (0, 0)),
            pl.BlockSpec((1, C), lambda i: (0, 0)),
            pl.BlockSpec((9 * C, 2 * C), lambda i: (0, 0)),
            pl.BlockSpec((1, 2 * C), lambda i: (0, 0)),
            pl.BlockSpec((9 * C, C), lambda i: (0, 0)),
            pl.BlockSpec((1, C), lambda i: (0, 0)),
            pl.BlockSpec((9 * C, C), lambda i: (0, 0)),
            pl.BlockSpec((1, C), lambda i: (0, 0)),
            pl.BlockSpec((2 * C, co), lambda i: (0, 0)),
            pl.BlockSpec((1, co), lambda i: (0, 0)),
        ],
        out_specs=pl.BlockSpec((1, co, h * w), lambda i: (i, 0, 0)),
        scratch_shapes=[pltpu.VMEM((R, C), bf16)] * 3
        + [pltpu.VMEM((Pc, 2 * C), bf16)],
        compiler_params=pltpu.CompilerParams(dimension_semantics=("parallel",)),
    )(x, mask, wm, bm[None, :], w1, b1, w2c, b2c[None, :], w2r, b2r[None, :],
      wh, bh)

    return out.reshape(n, co, h, w)
